# baseline (speedup 1.0000x reference)
import functools

import jax
import jax.numpy as jnp
from jax import lax
from jax.experimental import pallas as pl
from jax.experimental.pallas import tpu as pltpu
from jax.experimental.pallas import tpu_sc as plsc

B = 16384
DIM = 64
NC = 2
NS = 16
NW = NC * NS
BPW = B // NW
CHUNK = 128
NCH = BPW // CHUNK

_mesh = plsc.VectorSubcoreMesh(core_axis_name="c", subcore_axis_name="s")


@functools.partial(
    pl.kernel,
    out_type=jax.ShapeDtypeStruct((B,), jnp.float32),
    mesh=_mesh,
    compiler_params=pltpu.CompilerParams(
        needs_layout_passes=False, use_tc_tiling_on_sc=False),
    scratch_types=[
        pltpu.VMEM((3, NCH, CHUNK), jnp.int32),
        pltpu.VMEM((BPW, DIM), jnp.float32),
        pltpu.VMEM((BPW, DIM), jnp.float32),
        pltpu.VMEM((BPW, DIM), jnp.float32),
        pltpu.VMEM((16,), jnp.float32),
        pltpu.VMEM((BPW,), jnp.float32),
        pltpu.SemaphoreType.DMA,
    ],
)
def _score_kernel(xi_hbm, ent_hbm, rel_hbm, wb_hbm, out_hbm,
                  idx_v, h_v, r_v, t_v, wb_v, out_v, sem):
    wid = lax.axis_index("s") * NC + lax.axis_index("c")
    base = wid * BPW

    pltpu.sync_copy(xi_hbm.at[wid], idx_v)
    pltpu.sync_copy(wb_hbm, wb_v)

    copies = []
    for c in range(NCH):
        dst = pl.ds(c * CHUNK, CHUNK)
        copies.append(pltpu.async_copy(
            ent_hbm.at[idx_v.at[0, c]], h_v.at[dst], sem))
        copies.append(pltpu.async_copy(
            rel_hbm.at[idx_v.at[1, c]], r_v.at[dst], sem))
        copies.append(pltpu.async_copy(
            ent_hbm.at[idx_v.at[2, c]], t_v.at[dst], sem))
    for cp in copies:
        cp.wait()

    wbv = wb_v[...]
    w = wbv[0]
    b0 = wbv[1]

    def blk_body(i, carry):
        rows = lax.iota(jnp.int32, 16) + i * 16

        def d_body(dd, acc):
            cols = jnp.full((16,), dd, jnp.int32)
            hv = plsc.load_gather(h_v, [rows, cols])
            rv = plsc.load_gather(r_v, [rows, cols])
            tv = plsc.load_gather(t_v, [rows, cols])
            return acc + rv * (hv * tv)

        acc = lax.fori_loop(0, DIM, d_body, jnp.zeros((16,), jnp.float32))
        z = acc * w + b0
        out_v[pl.ds(i * 16, 16)] = 1.0 / (1.0 + jnp.exp(-z))
        return carry

    lax.fori_loop(0, BPW // 16, blk_body, 0)

    pltpu.sync_copy(out_v, out_hbm.at[pl.ds(base, BPW)])


def kernel(x, ent_embed, rel_embed, lin_w, lin_b):
    xi = x.astype(jnp.int32).T.reshape(3, NW, NCH, CHUNK).transpose(1, 0, 2, 3)
    wb = jnp.zeros((16,), jnp.float32).at[0].set(lin_w[0, 0]).at[1].set(lin_b[0])
    return _score_kernel(xi, ent_embed, rel_embed, wb)
